# baseline (speedup 1.0000x reference)
import functools
import math

import jax
import jax.numpy as jnp
from jax.experimental import pallas as pl
from jax.experimental.pallas import tpu as pltpu


_N_EXP = 8
_TOP_K = 2
_CAP_FACTOR = 1.25
_MIN_CAP = 4


def _routing_finish(logits_ref, route_ref, uc_ref, capacity):
    logits = logits_ref[:]
    T, E = logits.shape
    lane = jax.lax.broadcasted_iota(jnp.int32, (T, E), 1)

    m0 = jnp.max(logits, axis=1, keepdims=True)
    idx0 = jnp.min(jnp.where(logits == m0, lane, E), axis=1,
                   keepdims=True)
    masked = jnp.where(lane == idx0, -jnp.inf, logits)
    m1 = jnp.max(masked, axis=1, keepdims=True)
    idx1 = jnp.min(jnp.where(masked == m1, lane, E), axis=1,
                   keepdims=True)

    s = jnp.exp(m1 - m0)
    denom = 1.0 + s
    p0 = 1.0 / denom
    p1 = s / denom

    cnt0 = (lane == idx0).astype(jnp.float32)
    cnt1 = (lane == idx1).astype(jnp.float32)

    cnt = jnp.concatenate([cnt0, cnt1], axis=1).astype(jnp.bfloat16)
    row = jax.lax.broadcasted_iota(jnp.int32, (T, T), 0)
    col = jax.lax.broadcasted_iota(jnp.int32, (T, T), 1)
    tri = (row >= col).astype(jnp.bfloat16)
    csum = jax.lax.dot_general(
        tri, cnt, (((1,), (0,)), ((), ())),
        preferred_element_type=jnp.float32)
    csum0 = csum[:, :E]
    csum1 = csum[:, E:]
    total0 = csum0[T - 1:T, :]
    rank0_full = csum0 - cnt0
    rank1_full = total0 + csum1 - cnt1
    r0 = jnp.sum(rank0_full * cnt0, axis=1, keepdims=True)
    r1 = jnp.sum(rank1_full * cnt1, axis=1, keepdims=True)

    keep0 = (r0 < capacity).astype(jnp.float32)
    keep1 = (r1 < capacity).astype(jnp.float32)
    uc_ref[:] = jnp.sum(cnt0 * keep0 + cnt1 * keep1, axis=0, keepdims=True)

    zeros = jnp.zeros_like(p0)
    route_ref[:] = jnp.concatenate(
        [idx0.astype(jnp.float32), idx1.astype(jnp.float32),
         p0, p1, r0, r1, zeros, zeros], axis=1)


def _fused_kernel(x_ref, wg_ref, uc_ref, cb_ref, mask_ref,
                  logits_ref, route_ref, *, capacity, n_route, dispatch_tb):
    i = pl.program_id(0)

    @pl.when(i < n_route)
    def _route_step():
        chunk = x_ref.shape[0]
        logits_ref[pl.ds(i * chunk, chunk), :] = jax.lax.dot_general(
            x_ref[:], wg_ref[:], (((1,), (1,)), ((), ())),
            preferred_element_type=jnp.float32)

    @pl.when(i == n_route - 1)
    def _route_finish():
        _routing_finish(logits_ref, route_ref, uc_ref, capacity)

    @pl.when(i >= n_route)
    def _dispatch_step():
        j = i - n_route
        r = route_ref[pl.ds(j * dispatch_tb, dispatch_tb), :]
        Tb = dispatch_tb
        idx0 = r[:, 0:1].astype(jnp.int32)
        idx1 = r[:, 1:2].astype(jnp.int32)
        p0 = r[:, 2:3]
        p1 = r[:, 3:4]
        r0 = r[:, 4:5].astype(jnp.int32)
        r1 = r[:, 5:6].astype(jnp.int32)
        col = jax.lax.broadcasted_iota(jnp.int32, (Tb, capacity), 1)
        for e in range(_N_EXP):
            hit0 = idx0 == e
            hit1 = idx1 == e
            re = jnp.where(hit0, r0, jnp.where(hit1, r1, -1))
            pe = jnp.where(hit0, p0, p1)
            re = jnp.where(pe != 0.0, re, -1)
            cmp = col == re
            cb_ref[:, e, :] = jnp.where(cmp, pe, 0.0)
            mask_ref[:, e, :] = cmp


def kernel(x, w_g):
    Bx, Tx, H = x.shape
    num_tokens = Bx * Tx
    E = w_g.shape[0]
    capacity = int(max(math.floor(_TOP_K * _CAP_FACTOR * num_tokens / E),
                       _MIN_CAP))
    x_flat = x.reshape(num_tokens, H)

    Tc = 256
    Tb = 512
    n_route = num_tokens // Tc
    n_disp = num_tokens // Tb

    uc, cb, mask = pl.pallas_call(
        functools.partial(_fused_kernel, capacity=capacity,
                          n_route=n_route, dispatch_tb=Tb),
        grid=(n_route + n_disp,),
        in_specs=[
            pl.BlockSpec((Tc, H),
                         lambda i: (jnp.minimum(i, n_route - 1), 0)),
            pl.BlockSpec((E, H), lambda i: (0, 0)),
        ],
        out_specs=[
            pl.BlockSpec((1, E), lambda i: (0, 0)),
            pl.BlockSpec((Tb, E, capacity),
                         lambda i: (jnp.maximum(i - n_route, 0), 0, 0)),
            pl.BlockSpec((Tb, E, capacity),
                         lambda i: (jnp.maximum(i - n_route, 0), 0, 0)),
        ],
        out_shape=[
            jax.ShapeDtypeStruct((1, E), jnp.float32),
            jax.ShapeDtypeStruct((num_tokens, E, capacity), jnp.float32),
            jax.ShapeDtypeStruct((num_tokens, E, capacity), jnp.bool_),
        ],
        scratch_shapes=[
            pltpu.VMEM((num_tokens, E), jnp.float32),
            pltpu.VMEM((num_tokens, E), jnp.float32),
        ],
    )(x_flat, w_g)
    used_capacity = uc.reshape(E).astype(jnp.int32)
    return (used_capacity, cb, mask)


# speedup vs baseline: 1.0766x; 1.0766x over previous
import functools
import math

import jax
import jax.numpy as jnp
from jax.experimental import pallas as pl
from jax.experimental.pallas import tpu as pltpu


_N_EXP = 8
_TOP_K = 2
_CAP_FACTOR = 1.25
_MIN_CAP = 4


def _routing_finish(logits_ref, route_ref, uc_ref, capacity):
    lt = jnp.transpose(logits_ref[:])
    E, T = lt.shape
    slane = jax.lax.broadcasted_iota(jnp.int32, (E, T), 0)

    m0 = jnp.max(lt, axis=0, keepdims=True)
    idx0 = jnp.min(jnp.where(lt == m0, slane, E), axis=0, keepdims=True)
    masked = jnp.where(slane == idx0, -jnp.inf, lt)
    m1 = jnp.max(masked, axis=0, keepdims=True)
    idx1 = jnp.min(jnp.where(masked == m1, slane, E), axis=0,
                   keepdims=True)

    s = jnp.exp(m1 - m0)
    denom = 1.0 + s
    p0 = 1.0 / denom
    p1 = s / denom

    cnt0 = (slane == idx0).astype(jnp.bfloat16)
    cnt1 = (slane == idx1).astype(jnp.bfloat16)

    cnt16 = jnp.concatenate([cnt0, cnt1], axis=0)
    row = jax.lax.broadcasted_iota(jnp.int32, (T, T), 0)
    col = jax.lax.broadcasted_iota(jnp.int32, (T, T), 1)
    triu = (row <= col).astype(jnp.bfloat16)
    csum = jax.lax.dot_general(
        cnt16, triu, (((1,), (0,)), ((), ())),
        preferred_element_type=jnp.float32)
    csum0 = csum[:E, :]
    csum1 = csum[E:, :]
    total0 = csum0[:, T - 1:T]
    total1 = csum1[:, T - 1:T]
    cnt0f = cnt0.astype(jnp.float32)
    cnt1f = cnt1.astype(jnp.float32)
    rank0_full = csum0 - cnt0f
    rank1_full = total0 + csum1 - cnt1f
    r0 = jnp.sum(rank0_full * cnt0f, axis=0, keepdims=True)
    r1 = jnp.sum(rank1_full * cnt1f, axis=0, keepdims=True)

    uc_ref[:] = jnp.minimum(total0 + total1, float(capacity))

    zeros = jnp.zeros_like(p0)
    rt8 = jnp.concatenate(
        [idx0.astype(jnp.float32), idx1.astype(jnp.float32),
         p0, p1, r0, r1, zeros, zeros], axis=0)
    route_ref[:] = jnp.transpose(rt8)


def _fused_kernel(x_ref, wg_ref, uc_ref, cb_ref, mask_ref,
                  logits_ref, route_ref, *, capacity, n_route, dispatch_tb):
    i = pl.program_id(0)

    @pl.when(i < n_route)
    def _route_step():
        chunk = x_ref.shape[0]
        logits_ref[pl.ds(i * chunk, chunk), :] = jax.lax.dot_general(
            x_ref[:], wg_ref[:], (((1,), (1,)), ((), ())),
            preferred_element_type=jnp.float32)

    @pl.when(i == n_route - 1)
    def _route_finish():
        _routing_finish(logits_ref, route_ref, uc_ref, capacity)

    @pl.when(i >= n_route)
    def _dispatch_step():
        j = i - n_route
        r = route_ref[pl.ds(j * dispatch_tb, dispatch_tb), :]
        Tb = dispatch_tb
        idx0 = r[:, 0:1].astype(jnp.int32)
        idx1 = r[:, 1:2].astype(jnp.int32)
        p0 = r[:, 2:3]
        p1 = r[:, 3:4]
        r0 = r[:, 4:5].astype(jnp.int32)
        r1 = r[:, 5:6].astype(jnp.int32)
        col = jax.lax.broadcasted_iota(jnp.int32, (Tb, capacity), 1)
        for e in range(_N_EXP):
            hit0 = idx0 == e
            hit1 = idx1 == e
            re = jnp.where(hit0, r0, jnp.where(hit1, r1, -1))
            pe = jnp.where(hit0, p0, p1)
            re = jnp.where(pe != 0.0, re, -1)
            cmp = col == re
            cb_ref[:, e, :] = jnp.where(cmp, pe, 0.0)
            mask_ref[:, e, :] = cmp


def kernel(x, w_g):
    Bx, Tx, H = x.shape
    num_tokens = Bx * Tx
    E = w_g.shape[0]
    capacity = int(max(math.floor(_TOP_K * _CAP_FACTOR * num_tokens / E),
                       _MIN_CAP))
    x_flat = x.reshape(num_tokens, H)

    Tc = 256
    Tb = 256
    n_route = num_tokens // Tc
    n_disp = num_tokens // Tb

    uc, cb, mask = pl.pallas_call(
        functools.partial(_fused_kernel, capacity=capacity,
                          n_route=n_route, dispatch_tb=Tb),
        grid=(n_route + n_disp,),
        in_specs=[
            pl.BlockSpec((Tc, H),
                         lambda i: (jnp.minimum(i, n_route - 1), 0)),
            pl.BlockSpec((E, H), lambda i: (0, 0)),
        ],
        out_specs=[
            pl.BlockSpec((E, 1), lambda i: (0, 0)),
            pl.BlockSpec((Tb, E, capacity),
                         lambda i: (jnp.maximum(i - n_route, 0), 0, 0)),
            pl.BlockSpec((Tb, E, capacity),
                         lambda i: (jnp.maximum(i - n_route, 0), 0, 0)),
        ],
        out_shape=[
            jax.ShapeDtypeStruct((E, 1), jnp.float32),
            jax.ShapeDtypeStruct((num_tokens, E, capacity), jnp.float32),
            jax.ShapeDtypeStruct((num_tokens, E, capacity), jnp.bool_),
        ],
        scratch_shapes=[
            pltpu.VMEM((num_tokens, E), jnp.float32),
            pltpu.VMEM((num_tokens, E), jnp.float32),
        ],
    )(x_flat, w_g)
    used_capacity = uc.reshape(E).astype(jnp.int32)
    return (used_capacity, cb, mask)


# speedup vs baseline: 1.1007x; 1.0224x over previous
import functools
import math

import jax
import jax.numpy as jnp
from jax.experimental import pallas as pl
from jax.experimental.pallas import tpu as pltpu


_N_EXP = 8
_TOP_K = 2
_CAP_FACTOR = 1.25
_MIN_CAP = 4


def _routing_finish(logits_ref, route_ref, uc_ref, capacity):
    lt = jnp.transpose(logits_ref[:])
    E, T = lt.shape
    slane = jax.lax.broadcasted_iota(jnp.int32, (E, T), 0)

    m0 = jnp.max(lt, axis=0, keepdims=True)
    idx0 = jnp.min(jnp.where(lt == m0, slane, E), axis=0, keepdims=True)
    masked = jnp.where(slane == idx0, -jnp.inf, lt)
    m1 = jnp.max(masked, axis=0, keepdims=True)
    idx1 = jnp.min(jnp.where(masked == m1, slane, E), axis=0,
                   keepdims=True)

    s = jnp.exp(m1 - m0)
    denom = 1.0 + s
    p0 = 1.0 / denom
    p1 = s / denom

    cnt0 = (slane == idx0).astype(jnp.bfloat16)
    cnt1 = (slane == idx1).astype(jnp.bfloat16)

    cnt16 = jnp.concatenate([cnt0, cnt1], axis=0)
    row = jax.lax.broadcasted_iota(jnp.int32, (T, T), 0)
    col = jax.lax.broadcasted_iota(jnp.int32, (T, T), 1)
    triu = (row <= col).astype(jnp.bfloat16)
    csum = jax.lax.dot_general(
        cnt16, triu, (((1,), (0,)), ((), ())),
        preferred_element_type=jnp.float32)
    csum0 = csum[:E, :]
    csum1 = csum[E:, :]
    total0 = csum0[:, T - 1:T]
    total1 = csum1[:, T - 1:T]
    cnt0f = cnt0.astype(jnp.float32)
    cnt1f = cnt1.astype(jnp.float32)
    rank0_full = csum0 - cnt0f
    rank1_full = total0 + csum1 - cnt1f
    r0 = jnp.sum(rank0_full * cnt0f, axis=0, keepdims=True)
    r1 = jnp.sum(rank1_full * cnt1f, axis=0, keepdims=True)

    uc_ref[:] = jnp.minimum(total0 + total1, float(capacity))

    zeros = jnp.zeros_like(p0)
    rt8 = jnp.concatenate(
        [idx0.astype(jnp.float32), idx1.astype(jnp.float32),
         p0, p1, r0, r1, zeros, zeros], axis=0)
    route_ref[:] = jnp.transpose(rt8)


def _fused_kernel(x_ref, wg_ref, uc_ref, cb_ref, mask_ref,
                  logits_ref, route_ref, *, capacity, n_route, dispatch_tb):
    i = pl.program_id(0)

    @pl.when(i < n_route)
    def _route_step():
        chunk = x_ref.shape[0]
        logits_ref[pl.ds(i * chunk, chunk), :] = jax.lax.dot_general(
            x_ref[:], wg_ref[:], (((1,), (1,)), ((), ())),
            preferred_element_type=jnp.float32)

    @pl.when(i == n_route - 1)
    def _route_finish():
        _routing_finish(logits_ref, route_ref, uc_ref, capacity)

    @pl.when(i >= n_route)
    def _dispatch_step():
        j = i - n_route
        r = route_ref[pl.ds(j * dispatch_tb, dispatch_tb), :]
        Tb = dispatch_tb
        idx0 = r[:, 0:1].astype(jnp.int32)
        idx1 = r[:, 1:2].astype(jnp.int32)
        p0 = r[:, 2:3]
        p1 = r[:, 3:4]
        r0 = r[:, 4:5].astype(jnp.int32)
        r1 = r[:, 5:6].astype(jnp.int32)
        col = jax.lax.broadcasted_iota(jnp.int32, (Tb, capacity), 1)
        for e in range(_N_EXP):
            hit0 = idx0 == e
            hit1 = idx1 == e
            re = jnp.where(hit0, r0, jnp.where(hit1, r1, -1))
            pe = jnp.where(hit0, p0, p1)
            re = jnp.where(pe != 0.0, re, -1)
            cmp = col == re
            cb_ref[:, e, :] = jnp.where(cmp, pe, 0.0)
            mask_ref[:, e, :] = cmp


def kernel(x, w_g):
    Bx, Tx, H = x.shape
    num_tokens = Bx * Tx
    E = w_g.shape[0]
    capacity = int(max(math.floor(_TOP_K * _CAP_FACTOR * num_tokens / E),
                       _MIN_CAP))
    x_flat = x.reshape(num_tokens, H)

    Tc = 512
    Tb = 256
    n_route = num_tokens // Tc
    n_disp = num_tokens // Tb

    uc, cb, mask = pl.pallas_call(
        functools.partial(_fused_kernel, capacity=capacity,
                          n_route=n_route, dispatch_tb=Tb),
        grid=(n_route + n_disp,),
        in_specs=[
            pl.BlockSpec((Tc, H),
                         lambda i: (jnp.minimum(i, n_route - 1), 0)),
            pl.BlockSpec((E, H), lambda i: (0, 0)),
        ],
        out_specs=[
            pl.BlockSpec((E, 1), lambda i: (0, 0)),
            pl.BlockSpec((Tb, E, capacity),
                         lambda i: (jnp.maximum(i - n_route, 0), 0, 0)),
            pl.BlockSpec((Tb, E, capacity),
                         lambda i: (jnp.maximum(i - n_route, 0), 0, 0)),
        ],
        out_shape=[
            jax.ShapeDtypeStruct((E, 1), jnp.float32),
            jax.ShapeDtypeStruct((num_tokens, E, capacity), jnp.float32),
            jax.ShapeDtypeStruct((num_tokens, E, capacity), jnp.bool_),
        ],
        scratch_shapes=[
            pltpu.VMEM((num_tokens, E), jnp.float32),
            pltpu.VMEM((num_tokens, E), jnp.float32),
        ],
    )(x_flat, w_g)
    used_capacity = uc.reshape(E).astype(jnp.int32)
    return (used_capacity, cb, mask)


# speedup vs baseline: 1.1082x; 1.0068x over previous
import functools
import math

import jax
import jax.numpy as jnp
from jax.experimental import pallas as pl
from jax.experimental.pallas import tpu as pltpu


_N_EXP = 8
_TOP_K = 2
_CAP_FACTOR = 1.25
_MIN_CAP = 4


def _routing_finish(logits_ref, route_ref, uc_ref, capacity):
    lt = jnp.transpose(logits_ref[:])
    E, T = lt.shape
    slane = jax.lax.broadcasted_iota(jnp.int32, (E, T), 0)

    m0 = jnp.max(lt, axis=0, keepdims=True)
    idx0 = jnp.min(jnp.where(lt == m0, slane, E), axis=0, keepdims=True)
    masked = jnp.where(slane == idx0, -jnp.inf, lt)
    m1 = jnp.max(masked, axis=0, keepdims=True)
    idx1 = jnp.min(jnp.where(masked == m1, slane, E), axis=0,
                   keepdims=True)

    s = jnp.exp(m1 - m0)
    denom = 1.0 + s
    p0 = 1.0 / denom
    p1 = s / denom

    cnt0 = (slane == idx0).astype(jnp.bfloat16)
    cnt1 = (slane == idx1).astype(jnp.bfloat16)

    cnt16 = jnp.concatenate([cnt0, cnt1], axis=0)
    row = jax.lax.broadcasted_iota(jnp.int32, (T, T), 0)
    col = jax.lax.broadcasted_iota(jnp.int32, (T, T), 1)
    triu = (row <= col).astype(jnp.bfloat16)
    csum = jax.lax.dot_general(
        cnt16, triu, (((1,), (0,)), ((), ())),
        preferred_element_type=jnp.float32)
    csum0 = csum[:E, :]
    csum1 = csum[E:, :]
    total0 = csum0[:, T - 1:T]
    total1 = csum1[:, T - 1:T]
    cnt0f = cnt0.astype(jnp.float32)
    cnt1f = cnt1.astype(jnp.float32)
    rank0_full = csum0 - cnt0f
    rank1_full = total0 + csum1 - cnt1f
    r0 = jnp.sum(rank0_full * cnt0f, axis=0, keepdims=True)
    r1 = jnp.sum(rank1_full * cnt1f, axis=0, keepdims=True)

    uc_ref[:] = jnp.minimum(total0 + total1, float(capacity))

    zeros = jnp.zeros_like(p0)
    rt8 = jnp.concatenate(
        [idx0.astype(jnp.float32), idx1.astype(jnp.float32),
         p0, p1, r0, r1, zeros, zeros], axis=0)
    route_ref[:] = jnp.transpose(rt8)


def _fused_kernel(x_ref, wg_ref, uc_ref, cb_ref, mask_ref,
                  logits_ref, route_ref, *, capacity, n_route, dispatch_tb):
    i = pl.program_id(0)

    @pl.when(i < n_route)
    def _route_step():
        chunk = x_ref.shape[0]
        logits_ref[pl.ds(i * chunk, chunk), :] = jax.lax.dot_general(
            x_ref[:], wg_ref[:], (((1,), (1,)), ((), ())),
            preferred_element_type=jnp.float32)

    @pl.when(i == n_route - 1)
    def _route_finish():
        _routing_finish(logits_ref, route_ref, uc_ref, capacity)

    @pl.when(i >= n_route)
    def _dispatch_step():
        j = i - n_route
        r = route_ref[pl.ds(j * dispatch_tb, dispatch_tb), :]
        Tb = dispatch_tb
        idx0 = r[:, 0:1].astype(jnp.int32)
        idx1 = r[:, 1:2].astype(jnp.int32)
        p0 = r[:, 2:3]
        p1 = r[:, 3:4]
        r0 = r[:, 4:5].astype(jnp.int32)
        r1 = r[:, 5:6].astype(jnp.int32)
        col = jax.lax.broadcasted_iota(jnp.int32, (Tb, capacity), 1)
        for e in range(_N_EXP):
            hit0 = idx0 == e
            hit1 = idx1 == e
            re = jnp.where(hit0, r0, jnp.where(hit1, r1, -1))
            pe = jnp.where(hit0, p0, p1)
            re = jnp.where(pe != 0.0, re, -1)
            cmp = col == re
            cb_ref[:, e, :] = jnp.where(cmp, pe, 0.0)
            mask_ref[:, e, :] = cmp


def kernel(x, w_g):
    Bx, Tx, H = x.shape
    num_tokens = Bx * Tx
    E = w_g.shape[0]
    capacity = int(max(math.floor(_TOP_K * _CAP_FACTOR * num_tokens / E),
                       _MIN_CAP))
    x_flat = x.reshape(num_tokens, H)

    Tc = 1024
    Tb = 256
    n_route = num_tokens // Tc
    n_disp = num_tokens // Tb

    uc, cb, mask = pl.pallas_call(
        functools.partial(_fused_kernel, capacity=capacity,
                          n_route=n_route, dispatch_tb=Tb),
        grid=(n_route + n_disp,),
        in_specs=[
            pl.BlockSpec((Tc, H),
                         lambda i: (jnp.minimum(i, n_route - 1), 0)),
            pl.BlockSpec((E, H), lambda i: (0, 0)),
        ],
        out_specs=[
            pl.BlockSpec((E, 1), lambda i: (0, 0)),
            pl.BlockSpec((Tb, E, capacity),
                         lambda i: (jnp.maximum(i - n_route, 0), 0, 0)),
            pl.BlockSpec((Tb, E, capacity),
                         lambda i: (jnp.maximum(i - n_route, 0), 0, 0)),
        ],
        out_shape=[
            jax.ShapeDtypeStruct((E, 1), jnp.float32),
            jax.ShapeDtypeStruct((num_tokens, E, capacity), jnp.float32),
            jax.ShapeDtypeStruct((num_tokens, E, capacity), jnp.bool_),
        ],
        scratch_shapes=[
            pltpu.VMEM((num_tokens, E), jnp.float32),
            pltpu.VMEM((num_tokens, E), jnp.float32),
        ],
    )(x_flat, w_g)
    used_capacity = uc.reshape(E).astype(jnp.int32)
    return (used_capacity, cb, mask)
